# baseline (speedup 1.0000x reference)
import functools

import jax
import jax.numpy as jnp
from jax.experimental import pallas as pl
from jax.experimental.pallas import tpu as pltpu

NUM_HEAD = 8
DH = 8
NUM_LAYER = 2
DIM = 64
ROW_BLK = 512


def _proj_body(x_ref, w_ref, b_ref, o_ref):
    o_ref[...] = jax.lax.dot_general(
        x_ref[...], w_ref[...], (((1,), (1,)), ((), ())),
        preferred_element_type=jnp.float32) + b_ref[...]


def _tc_proj(x, w, b):
    n = x.shape[0]
    assert n % ROW_BLK == 0
    return pl.pallas_call(
        _proj_body,
        grid=(n // ROW_BLK,),
        in_specs=[
            pl.BlockSpec((ROW_BLK, DIM), lambda i: (i, 0)),
            pl.BlockSpec((DIM, DIM), lambda i: (0, 0)),
            pl.BlockSpec((1, DIM), lambda i: (0, 0)),
        ],
        out_specs=pl.BlockSpec((ROW_BLK, DIM), lambda i: (i, 0)),
        out_shape=jax.ShapeDtypeStruct((n, DIM), jnp.float32),
    )(x, w, b.reshape(1, DIM))


def _ent_layer(le, lr, h, r, t, Racc, fdiv, p, n):
    aw = p['attn_w']
    W1, W2, W3 = aw[:, :64], aw[:, 64:128], aw[:, 128:192]
    ab = p['attn_b']
    gw = p['aggr_w']
    Ga, Gb = gw[:, :64], gw[:, 64:]
    gb = p['aggr_b']
    av = p['attn_vec'].reshape(NUM_HEAD, DH)

    A_t = _tc_proj(le, W1, jnp.zeros((DIM,), jnp.float32))
    A_h = _tc_proj(le, W2, ab)
    A_r = _tc_proj(lr, W3, ab)
    M_h = _tc_proj(le, Ga, gb)
    M_r = _tc_proj(lr, Gb, gb)
    S3 = _tc_proj(Racc, W3, jnp.zeros((DIM,), jnp.float32)) / fdiv
    Sm = _tc_proj(Racc, Gb, jnp.zeros((DIM,), jnp.float32)) / fdiv

    pre_self = A_t + (A_h - ab) + S3 + ab
    raw_self = (jnp.tanh(pre_self).reshape(n, NUM_HEAD, DH) * av).sum(-1)
    av_self = jnp.exp(raw_self)
    msg_self = ((M_h - gb) + Sm + gb).reshape(n, NUM_HEAD, DH)
    asum = av_self
    outnum = av_self[:, :, None] * msg_self

    pre_e = A_t[t] + A_h[h] + A_r[r] - ab
    raw_e = (jnp.tanh(pre_e).reshape(-1, NUM_HEAD, DH) * av).sum(-1)
    av_e = jnp.exp(raw_e)
    msg_e = (M_h[h] + M_r[r] - gb).reshape(-1, NUM_HEAD, DH)
    asum = asum.at[t].add(av_e)
    outnum = outnum.at[t].add(av_e[:, :, None] * msg_e)

    return (outnum / (asum[:, :, None] + 1e-16)).reshape(n, NUM_HEAD * DH)


def _rel_layer(lr, h, t, p, n):
    aw = p['attn_w']
    W1, W2 = aw[:, :64], aw[:, 64:]
    av = p['attn_vec'].reshape(NUM_HEAD, DH)
    A1 = _tc_proj(lr, W1, jnp.zeros((DIM,), jnp.float32))
    A2 = _tc_proj(lr, W2, p['attn_b'])
    M = _tc_proj(lr, p['aggr_w'], p['aggr_b'])

    pre = A1[h] + A2[t]
    raw = (jnp.tanh(pre).reshape(-1, NUM_HEAD, DH) * av).sum(-1)
    av_e = jnp.exp(raw)
    msg = M[t].reshape(-1, NUM_HEAD, DH)
    asum = jnp.zeros((n, NUM_HEAD), jnp.float32).at[h].add(av_e)
    outnum = jnp.zeros((n, NUM_HEAD, DH), jnp.float32).at[h].add(av_e[:, :, None] * msg)
    return (outnum / (asum[:, :, None] + 1e-16)).reshape(n, NUM_HEAD * DH)


def kernel(emb_ent, emb_rel, triplets, relation_triplets, params):
    n_ent = emb_ent.shape[0]
    n_rel = emb_rel.shape[0]
    N = 50176
    le = jnp.zeros((N, DIM), jnp.float32).at[:n_ent].set(emb_ent)
    lrr = jnp.zeros((N, DIM), jnp.float32).at[:n_rel].set(emb_rel)
    h, r, t = triplets[:, 0], triplets[:, 1], triplets[:, 2]
    rh, rt = relation_triplets[:, 0], relation_triplets[:, 1]

    le = _tc_proj(le, params['ent_proj1_w'], params['ent_proj1_b'])
    lrr = _tc_proj(lrr, params['rel_proj1_w'], params['rel_proj1_b'])
    for i in range(NUM_LAYER):
        res = _tc_proj(lrr, params['res_rel'][i]['w'], params['res_rel'][i]['b'])
        lrr = jnp.tanh(_rel_layer(lrr, rh, rt, params['rel_layers'][i], N) + res)

    freq = jnp.zeros((N,), jnp.float32).at[t].add(1.0)
    Racc = jnp.zeros((N, DIM), jnp.float32).at[t].add(lrr[r])
    fdiv = freq[:, None] + 1e-16

    for i in range(NUM_LAYER):
        res = _tc_proj(le, params['res_ent'][i]['w'], params['res_ent'][i]['b'])
        le = jnp.tanh(_ent_layer(le, lrr, h, r, t, Racc, fdiv,
                                 params['ent_layers'][i], N) + res)

    out_ent = _tc_proj(le, params['ent_proj2_w'], params['ent_proj2_b'])[:n_ent]
    out_rel = _tc_proj(lrr, params['rel_proj2_w'], params['rel_proj2_b'])[:n_rel]
    return out_ent, out_rel
